# Initial kernel scaffold; baseline (speedup 1.0000x reference)
import jax
import jax.numpy as jnp
from jax.experimental import pallas as pl


def kernel(uid_batch, mid_batch, cat_batch, mid_his_batch, cat_his_batch, noclk_mid_batch, noclk_cat_batch, uid_table, mid_table, cat_table):
    raise NotImplementedError("write your pallas kernel here")


# speedup vs baseline: 9.4984x; 9.4984x over previous
import functools

import jax
import jax.numpy as jnp
from jax import lax
from jax.experimental import pallas as pl
from jax.experimental.pallas import tpu as pltpu
from jax.experimental.pallas import tpu_sc as plsc

_B = 4096
_L = 50
_NEG = 5
_E = 32
_NIDX = 1 + _L + _L * _NEG
_P = 304
_NW = 32
_BW = _B // _NW
_R = 2
_G = _BW // _R

_mesh = plsc.VectorSubcoreMesh(core_axis_name="c", subcore_axis_name="s")


@functools.partial(
    pl.kernel,
    out_type=jax.ShapeDtypeStruct((_B, 7 * _E), jnp.float32),
    mesh=_mesh,
    scratch_types=[
        pltpu.VMEM((_BW,), jnp.int32),
        pltpu.VMEM((_BW, _E), jnp.float32),
        pltpu.VMEM((_R * _P,), jnp.int32),
        pltpu.VMEM((_R * _P,), jnp.int32),
        pltpu.VMEM((_R * _P, _E), jnp.float32),
        pltpu.VMEM((_R * _P, _E), jnp.float32),
        pltpu.VMEM((_BW, 7 * _E), jnp.float32),
        pltpu.SemaphoreType.DMA,
        pltpu.SemaphoreType.DMA,
    ],
    compiler_params=pltpu.CompilerParams(use_tc_tiling_on_sc=False),
)
def _embed_kernel(midx_hbm, catx_hbm, uid_hbm, uid_table, mid_table, cat_table,
                  out_hbm, uidx_v, ubuf, midx_v, cidx_v, mbuf, cbuf, stage,
                  sem_m, sem_c):
    wid = lax.axis_index("s") * 2 + lax.axis_index("c")
    base = wid * _BW

    pltpu.sync_copy(uid_hbm.at[pl.ds(base, _BW)], uidx_v)
    pltpu.async_copy(uid_table.at[uidx_v], ubuf, sem_m).wait()

    def chunk(g, carry):
        r0 = g * _R
        off = (base + r0) * _P
        pltpu.sync_copy(midx_hbm.at[pl.ds(off, _R * _P)], midx_v)
        pltpu.sync_copy(catx_hbm.at[pl.ds(off, _R * _P)], cidx_v)
        cm = pltpu.async_copy(mid_table.at[midx_v], mbuf, sem_m)
        cc = pltpu.async_copy(cat_table.at[cidx_v], cbuf, sem_c)
        cm.wait()
        cc.wait()
        for r in range(_R):
            p = r * _P
            row = r0 + r
            for h in range(2):
                sl = pl.ds(h * 16, 16)
                stage[row, pl.ds(0 * _E + h * 16, 16)] = ubuf[row, sl]
                stage[row, pl.ds(1 * _E + h * 16, 16)] = mbuf[p, sl]
                stage[row, pl.ds(2 * _E + h * 16, 16)] = cbuf[p, sl]

            def seg(lo, hi):
                z = jnp.zeros((16,), jnp.float32)

                def body(j, c):
                    return (c[0] + mbuf[p + j, pl.ds(0, 16)],
                            c[1] + mbuf[p + j, pl.ds(16, 16)],
                            c[2] + cbuf[p + j, pl.ds(0, 16)],
                            c[3] + cbuf[p + j, pl.ds(16, 16)])

                return lax.fori_loop(lo, hi, body, (z, z, z, z))

            m0, m1, c0, c1 = seg(1, 1 + _L)
            stage[row, pl.ds(3 * _E, 16)] = m0
            stage[row, pl.ds(3 * _E + 16, 16)] = m1
            stage[row, pl.ds(4 * _E, 16)] = c0
            stage[row, pl.ds(4 * _E + 16, 16)] = c1
            m0, m1, c0, c1 = seg(1 + _L, _NIDX)
            stage[row, pl.ds(5 * _E, 16)] = m0
            stage[row, pl.ds(5 * _E + 16, 16)] = m1
            stage[row, pl.ds(6 * _E, 16)] = c0
            stage[row, pl.ds(6 * _E + 16, 16)] = c1
        return carry

    lax.fori_loop(0, _G, chunk, 0)
    pltpu.sync_copy(stage, out_hbm.at[pl.ds(base, _BW)])


def kernel(uid_batch, mid_batch, cat_batch, mid_his_batch, cat_his_batch,
           noclk_mid_batch, noclk_cat_batch, uid_table, mid_table, cat_table):
    midx = jnp.concatenate(
        [mid_batch[:, None], mid_his_batch,
         noclk_mid_batch.reshape(_B, _L * _NEG)], axis=1)
    catx = jnp.concatenate(
        [cat_batch[:, None], cat_his_batch,
         noclk_cat_batch.reshape(_B, _L * _NEG)], axis=1)
    pad = ((0, 0), (0, _P - _NIDX))
    midx = jnp.pad(midx, pad, mode="wrap").reshape(_B * _P)
    catx = jnp.pad(catx, pad, mode="wrap").reshape(_B * _P)
    return _embed_kernel(midx, catx, uid_batch, uid_table, mid_table,
                         cat_table)


# speedup vs baseline: 10.8867x; 1.1462x over previous
import functools

import jax
import jax.numpy as jnp
from jax import lax
from jax.experimental import pallas as pl
from jax.experimental.pallas import tpu as pltpu
from jax.experimental.pallas import tpu_sc as plsc

_B = 4096
_L = 50
_NEG = 5
_E = 32
_NIDX = 1 + _L + _L * _NEG
_P = 304
_NW = 32
_BW = _B // _NW
_R = 2
_RP = _R * _P
_G = _BW // _R

_mesh = plsc.VectorSubcoreMesh(core_axis_name="c", subcore_axis_name="s")


@functools.partial(
    pl.kernel,
    out_type=jax.ShapeDtypeStruct((_B, 7 * _E), jnp.float32),
    mesh=_mesh,
    scratch_types=[
        pltpu.VMEM((_BW,), jnp.int32),
        pltpu.VMEM((_BW, _E), jnp.float32),
        pltpu.VMEM((2, 2, _RP), jnp.int32),
        pltpu.VMEM((2, _RP, _E), jnp.float32),
        pltpu.VMEM((2, _RP, _E), jnp.float32),
        pltpu.VMEM((_BW, 7 * _E), jnp.float32),
        pltpu.SemaphoreType.DMA,
        pltpu.SemaphoreType.DMA,
        pltpu.SemaphoreType.DMA,
        pltpu.SemaphoreType.DMA,
    ],
    compiler_params=pltpu.CompilerParams(use_tc_tiling_on_sc=False),
)
def _embed_kernel(idx_hbm, uid_hbm, uid_table, mid_table, cat_table,
                  out_hbm, uidx_v, ubuf, idx_v, mbuf, cbuf, stage,
                  sem_m0, sem_m1, sem_c0, sem_c1):
    wid = lax.axis_index("s") * 2 + lax.axis_index("c")
    base = wid * _BW
    chunk0 = wid * _G
    sem_m = (sem_m0, sem_m1)
    sem_c = (sem_c0, sem_c1)

    pltpu.sync_copy(uid_hbm.at[pl.ds(base, _BW)], uidx_v)
    u_cp = pltpu.async_copy(uid_table.at[uidx_v], ubuf, sem_m0)

    def fetch(g, slot):
        pltpu.sync_copy(idx_hbm.at[chunk0 + g], idx_v.at[slot])
        pltpu.async_copy(mid_table.at[idx_v.at[slot, 0]], mbuf.at[slot],
                         sem_m[slot])
        pltpu.async_copy(cat_table.at[idx_v.at[slot, 1]], cbuf.at[slot],
                         sem_c[slot])

    def consume(g, slot):
        pltpu.make_async_copy(mid_table.at[idx_v.at[slot, 0]], mbuf.at[slot],
                              sem_m[slot]).wait()
        pltpu.make_async_copy(cat_table.at[idx_v.at[slot, 1]], cbuf.at[slot],
                              sem_c[slot]).wait()
        mb = mbuf.at[slot]
        cb = cbuf.at[slot]
        for r in range(_R):
            p = r * _P
            row = g * _R + r
            for h in range(2):
                sl = pl.ds(h * 16, 16)
                stage[row, pl.ds(0 * _E + h * 16, 16)] = ubuf[row, sl]
                stage[row, pl.ds(1 * _E + h * 16, 16)] = mb[p, sl]
                stage[row, pl.ds(2 * _E + h * 16, 16)] = cb[p, sl]

            def seg(lo, hi, unroll):
                z = jnp.zeros((16,), jnp.float32)

                def body(j, c):
                    return (c[0] + mb[p + j, pl.ds(0, 16)],
                            c[1] + mb[p + j, pl.ds(16, 16)],
                            c[2] + cb[p + j, pl.ds(0, 16)],
                            c[3] + cb[p + j, pl.ds(16, 16)])

                return lax.fori_loop(lo, hi, body, (z, z, z, z),
                                     unroll=unroll)

            m0, m1, c0, c1 = seg(1, 1 + _L, 10)
            stage[row, pl.ds(3 * _E, 16)] = m0
            stage[row, pl.ds(3 * _E + 16, 16)] = m1
            stage[row, pl.ds(4 * _E, 16)] = c0
            stage[row, pl.ds(4 * _E + 16, 16)] = c1
            m0, m1, c0, c1 = seg(1 + _L, _NIDX, 10)
            stage[row, pl.ds(5 * _E, 16)] = m0
            stage[row, pl.ds(5 * _E + 16, 16)] = m1
            stage[row, pl.ds(6 * _E, 16)] = c0
            stage[row, pl.ds(6 * _E + 16, 16)] = c1

    u_cp.wait()
    fetch(0, 0)

    def step(i, carry):
        for par in range(2):
            g = 2 * i + par
            gn = jnp.minimum(g + 1, _G - 1)
            fetch(gn, 1 - par)
            consume(g, par)
        return carry

    lax.fori_loop(0, _G // 2, step, 0)
    pltpu.make_async_copy(mid_table.at[idx_v.at[0, 0]], mbuf.at[0],
                          sem_m0).wait()
    pltpu.make_async_copy(cat_table.at[idx_v.at[0, 1]], cbuf.at[0],
                          sem_c0).wait()

    pltpu.sync_copy(stage, out_hbm.at[pl.ds(base, _BW)])


def kernel(uid_batch, mid_batch, cat_batch, mid_his_batch, cat_his_batch,
           noclk_mid_batch, noclk_cat_batch, uid_table, mid_table, cat_table):
    midx = jnp.concatenate(
        [mid_batch[:, None], mid_his_batch,
         noclk_mid_batch.reshape(_B, _L * _NEG)], axis=1)
    catx = jnp.concatenate(
        [cat_batch[:, None], cat_his_batch,
         noclk_cat_batch.reshape(_B, _L * _NEG)], axis=1)
    pad = ((0, 0), (0, _P - _NIDX))
    midx = jnp.pad(midx, pad, mode="wrap").reshape(_B // _R, _RP)
    catx = jnp.pad(catx, pad, mode="wrap").reshape(_B // _R, _RP)
    idx = jnp.stack([midx, catx], axis=1)
    return _embed_kernel(idx, uid_batch, uid_table, mid_table, cat_table)
